# baseline (speedup 1.0000x reference)
import jax
import jax.numpy as jnp
from jax.experimental import pallas as pl
from jax.experimental.pallas import tpu as pltpu

_GRID = 12
_LANES = 512
_CH = 8
_MAIN_ROWS = (16 * 3 * 96 * 320) // _LANES
_VERT_ROWS = (16 * 9 * 96 * 320) // _LANES
_MAIN_BLK = _MAIN_ROWS // _GRID
_VERT_BLK = _VERT_ROWS // _GRID

_CLIP = 9.210240366975849


def _chunk_terms(x, t):
    xc = jnp.clip(x, -_CLIP, _CLIP)
    e = jnp.exp(-xc)
    r = 1.0 / (1.0 + e)
    ln1pe = jnp.log1p(e)
    log_s = -ln1pe
    log_1ms = -xc - ln1pe
    omp = e * r
    pos = t >= 0.9999
    pos_term = log_s * (omp * omp)
    omt = 1.0 - t
    omt2 = omt * omt
    neg_term = log_1ms * (r * r) * (omt2 * omt2)
    loss = jnp.where(pos, pos_term, neg_term)
    posf = jnp.where(pos, 1.0, 0.0)
    return loss, posf


def _accumulate_tile(x_ref, t_ref, nrows, acc_l_ref, acc_p_ref, first):
    def step(k, carry):
        al, ap = carry
        sl = pl.ds(k * _CH, _CH)
        loss, posf = _chunk_terms(x_ref[sl, :], t_ref[sl, :])
        return al + loss, ap + posf

    l0, p0 = _chunk_terms(x_ref[pl.ds(0, _CH), :], t_ref[pl.ds(0, _CH), :])
    al, ap = jax.lax.fori_loop(1, nrows // _CH, step, (l0, p0))
    if first:
        acc_l_ref[...] = al
        acc_p_ref[...] = ap
    else:
        acc_l_ref[...] += al
        acc_p_ref[...] += ap


def _body(ml_ref, mm_ref, vl_ref, vm_ref, out_ref,
          macc_l, macc_p, vacc_l, vacc_p):
    i = pl.program_id(0)

    @pl.when(i == 0)
    def _first():
        _accumulate_tile(ml_ref, mm_ref, _MAIN_BLK, macc_l, macc_p, True)
        _accumulate_tile(vl_ref, vm_ref, _VERT_BLK, vacc_l, vacc_p, True)

    @pl.when(i > 0)
    def _rest():
        _accumulate_tile(ml_ref, mm_ref, _MAIN_BLK, macc_l, macc_p, False)
        _accumulate_tile(vl_ref, vm_ref, _VERT_BLK, vacc_l, vacc_p, False)

    @pl.when(i == _GRID - 1)
    def _finalize():
        ms = jnp.sum(macc_l[...])
        mp = jnp.sum(macc_p[...])
        vs = jnp.sum(vacc_l[...])
        vp = jnp.sum(vacc_p[...])
        main_loss = ms / jnp.maximum(mp, 1.0)
        vert_loss = vs / jnp.maximum(vp, 1.0)
        out_ref[0, 0] = -(main_loss + vert_loss)


def kernel(main_kf_logits, main_kf_mask, vertex_kf_logits, vertex_kf_mask):
    ml = main_kf_logits.reshape(_MAIN_ROWS, _LANES)
    mm = main_kf_mask.reshape(_MAIN_ROWS, _LANES)
    vl = vertex_kf_logits.reshape(_VERT_ROWS, _LANES)
    vm = vertex_kf_mask.reshape(_VERT_ROWS, _LANES)

    main_spec = pl.BlockSpec((_MAIN_BLK, _LANES), lambda i: (i, 0))
    vert_spec = pl.BlockSpec((_VERT_BLK, _LANES), lambda i: (i, 0))

    out = pl.pallas_call(
        _body,
        grid=(_GRID,),
        in_specs=[main_spec, main_spec, vert_spec, vert_spec],
        out_specs=pl.BlockSpec(memory_space=pltpu.SMEM),
        out_shape=jax.ShapeDtypeStruct((1, 1), jnp.float32),
        scratch_shapes=[
            pltpu.VMEM((_CH, _LANES), jnp.float32),
            pltpu.VMEM((_CH, _LANES), jnp.float32),
            pltpu.VMEM((_CH, _LANES), jnp.float32),
            pltpu.VMEM((_CH, _LANES), jnp.float32),
        ],
        compiler_params=pltpu.CompilerParams(
            dimension_semantics=("arbitrary",),
        ),
    )(ml, mm, vl, vm)
    return out[0, 0]


# speedup vs baseline: 1.1909x; 1.1909x over previous
import jax
import jax.numpy as jnp
from jax.experimental import pallas as pl
from jax.experimental.pallas import tpu as pltpu

_GRID = 12
_LANES = 512
_CH = 8
_N_ACC = 4
_MAIN_ROWS = (16 * 3 * 96 * 320) // _LANES
_VERT_ROWS = (16 * 9 * 96 * 320) // _LANES
_MAIN_BLK = _MAIN_ROWS // _GRID
_VERT_BLK = _VERT_ROWS // _GRID

_LOG2E = 1.4426950408889634
_LN2 = 0.6931471805599453
_CLIP2 = 9.210240366975849 * _LOG2E


def _chunk_terms(x, t):
    x2 = jnp.clip(x * _LOG2E, -_CLIP2, _CLIP2)
    e = jnp.exp2(-x2)
    ope = 1.0 + e
    r = 1.0 / ope
    lg = jnp.log2(ope)
    omp = e * r
    pos = t >= 0.9999
    pos_term = lg * (omp * omp)
    omt = 1.0 - t
    omt2 = omt * omt
    neg_term = (x2 + lg) * (r * r) * (omt2 * omt2)
    loss = jnp.where(pos, pos_term, neg_term)
    posf = jnp.where(pos, 1.0, 0.0)
    return loss, posf


def _tile_sums(x_ref, t_ref, nchunks):
    accs = []
    for k in range(nchunks):
        rows = slice(k * _CH, (k + 1) * _CH)
        loss, posf = _chunk_terms(x_ref[rows, :], t_ref[rows, :])
        if k < _N_ACC:
            accs.append([loss, posf])
        else:
            a = accs[k % _N_ACC]
            a[0] += loss
            a[1] += posf
    while len(accs) > 1:
        nxt = []
        for j in range(0, len(accs) - 1, 2):
            nxt.append([accs[j][0] + accs[j + 1][0],
                        accs[j][1] + accs[j + 1][1]])
        if len(accs) % 2:
            nxt.append(accs[-1])
        accs = nxt
    return accs[0]


def _body(ml_ref, mm_ref, vl_ref, vm_ref, out_ref,
          macc_l, macc_p, vacc_l, vacc_p):
    i = pl.program_id(0)
    m_l, m_p = _tile_sums(ml_ref, mm_ref, _MAIN_BLK // _CH)
    v_l, v_p = _tile_sums(vl_ref, vm_ref, _VERT_BLK // _CH)

    @pl.when(i == 0)
    def _init():
        macc_l[...] = m_l
        macc_p[...] = m_p
        vacc_l[...] = v_l
        vacc_p[...] = v_p

    @pl.when(i > 0)
    def _accum():
        macc_l[...] += m_l
        macc_p[...] += m_p
        vacc_l[...] += v_l
        vacc_p[...] += v_p

    @pl.when(i == _GRID - 1)
    def _finalize():
        ms = jnp.sum(macc_l[...]) * _LN2
        mp = jnp.sum(macc_p[...])
        vs = jnp.sum(vacc_l[...]) * _LN2
        vp = jnp.sum(vacc_p[...])
        main_loss = ms / jnp.maximum(mp, 1.0)
        vert_loss = vs / jnp.maximum(vp, 1.0)
        out_ref[0, 0] = main_loss + vert_loss


def kernel(main_kf_logits, main_kf_mask, vertex_kf_logits, vertex_kf_mask):
    ml = main_kf_logits.reshape(_MAIN_ROWS, _LANES)
    mm = main_kf_mask.reshape(_MAIN_ROWS, _LANES)
    vl = vertex_kf_logits.reshape(_VERT_ROWS, _LANES)
    vm = vertex_kf_mask.reshape(_VERT_ROWS, _LANES)

    main_spec = pl.BlockSpec((_MAIN_BLK, _LANES), lambda i: (i, 0))
    vert_spec = pl.BlockSpec((_VERT_BLK, _LANES), lambda i: (i, 0))

    out = pl.pallas_call(
        _body,
        grid=(_GRID,),
        in_specs=[main_spec, main_spec, vert_spec, vert_spec],
        out_specs=pl.BlockSpec(memory_space=pltpu.SMEM),
        out_shape=jax.ShapeDtypeStruct((1, 1), jnp.float32),
        scratch_shapes=[
            pltpu.VMEM((_CH, _LANES), jnp.float32),
            pltpu.VMEM((_CH, _LANES), jnp.float32),
            pltpu.VMEM((_CH, _LANES), jnp.float32),
            pltpu.VMEM((_CH, _LANES), jnp.float32),
        ],
        compiler_params=pltpu.CompilerParams(
            dimension_semantics=("arbitrary",),
        ),
    )(ml, mm, vl, vm)
    return out[0, 0]


# speedup vs baseline: 3.7872x; 3.1800x over previous
import jax
import jax.numpy as jnp
from jax.experimental import pallas as pl
from jax.experimental.pallas import tpu as pltpu

_GRID = 12
_H, _W = 96, 320
_CH = 8
_N_ACC = 4
_MAIN_SLICES = 16 * 3
_VERT_SLICES = 16 * 9
_MAIN_BLK = _MAIN_SLICES // _GRID
_VERT_BLK = _VERT_SLICES // _GRID

_LOG2E = 1.4426950408889634
_LN2 = 0.6931471805599453
_CLIP2 = 9.210240366975849 * _LOG2E


def _chunk_terms(x, t):
    x2 = jnp.clip(x * _LOG2E, -_CLIP2, _CLIP2)
    e = jnp.exp2(-x2)
    ope = 1.0 + e
    r = 1.0 / ope
    lg = jnp.log2(ope)
    omp = e * r
    pos = t >= 0.9999
    pos_term = lg * (omp * omp)
    omt = 1.0 - t
    omt2 = omt * omt
    neg_term = (x2 + lg) * (r * r) * (omt2 * omt2)
    loss = jnp.where(pos, pos_term, neg_term)
    posf = jnp.where(pos, 1.0, 0.0)
    return loss, posf


def _tile_sums(x_ref, t_ref, nslices):
    accs = []
    k = 0
    for s in range(nslices):
        for r0 in range(0, _H, _CH):
            rows = slice(r0, r0 + _CH)
            loss, posf = _chunk_terms(x_ref[s, rows, :], t_ref[s, rows, :])
            if k < _N_ACC:
                accs.append([loss, posf])
            else:
                a = accs[k % _N_ACC]
                a[0] += loss
                a[1] += posf
            k += 1
    while len(accs) > 1:
        nxt = []
        for j in range(0, len(accs) - 1, 2):
            nxt.append([accs[j][0] + accs[j + 1][0],
                        accs[j][1] + accs[j + 1][1]])
        if len(accs) % 2:
            nxt.append(accs[-1])
        accs = nxt
    return accs[0]


def _body(ml_ref, mm_ref, vl_ref, vm_ref, out_ref,
          macc_l, macc_p, vacc_l, vacc_p):
    i = pl.program_id(0)
    m_l, m_p = _tile_sums(ml_ref, mm_ref, _MAIN_BLK)
    v_l, v_p = _tile_sums(vl_ref, vm_ref, _VERT_BLK)

    @pl.when(i == 0)
    def _init():
        macc_l[...] = m_l
        macc_p[...] = m_p
        vacc_l[...] = v_l
        vacc_p[...] = v_p

    @pl.when(i > 0)
    def _accum():
        macc_l[...] += m_l
        macc_p[...] += m_p
        vacc_l[...] += v_l
        vacc_p[...] += v_p

    @pl.when(i == _GRID - 1)
    def _finalize():
        ms = jnp.sum(macc_l[...]) * _LN2
        mp = jnp.sum(macc_p[...])
        vs = jnp.sum(vacc_l[...]) * _LN2
        vp = jnp.sum(vacc_p[...])
        main_loss = ms / jnp.maximum(mp, 1.0)
        vert_loss = vs / jnp.maximum(vp, 1.0)
        out_ref[0, 0] = main_loss + vert_loss


def kernel(main_kf_logits, main_kf_mask, vertex_kf_logits, vertex_kf_mask):
    ml = main_kf_logits.reshape(_MAIN_SLICES, _H, _W)
    mm = main_kf_mask.reshape(_MAIN_SLICES, _H, _W)
    vl = vertex_kf_logits.reshape(_VERT_SLICES, _H, _W)
    vm = vertex_kf_mask.reshape(_VERT_SLICES, _H, _W)

    main_spec = pl.BlockSpec((_MAIN_BLK, _H, _W), lambda i: (i, 0, 0))
    vert_spec = pl.BlockSpec((_VERT_BLK, _H, _W), lambda i: (i, 0, 0))

    out = pl.pallas_call(
        _body,
        grid=(_GRID,),
        in_specs=[main_spec, main_spec, vert_spec, vert_spec],
        out_specs=pl.BlockSpec(memory_space=pltpu.SMEM),
        out_shape=jax.ShapeDtypeStruct((1, 1), jnp.float32),
        scratch_shapes=[
            pltpu.VMEM((_CH, _W), jnp.float32),
            pltpu.VMEM((_CH, _W), jnp.float32),
            pltpu.VMEM((_CH, _W), jnp.float32),
            pltpu.VMEM((_CH, _W), jnp.float32),
        ],
        compiler_params=pltpu.CompilerParams(
            dimension_semantics=("arbitrary",),
        ),
    )(ml, mm, vl, vm)
    return out[0, 0]


# speedup vs baseline: 3.9227x; 1.0358x over previous
import jax
import jax.numpy as jnp
from jax.experimental import pallas as pl
from jax.experimental.pallas import tpu as pltpu

_GRID = 12
_H, _W = 96, 320
_CH = 8
_N_ACC = 4
_MAIN_SLICES = 16 * 3
_VERT_SLICES = 16 * 9
_MAIN_BLK = _MAIN_SLICES // _GRID
_VERT_BLK = _VERT_SLICES // _GRID

_LOG2E = 1.4426950408889634
_LN2 = 0.6931471805599453
_CLIP2 = 9.210240366975849 * _LOG2E


def _chunk_terms(x, t):
    x2 = jnp.clip(x * _LOG2E, -_CLIP2, _CLIP2)
    e = jnp.exp2(-x2)
    ope = 1.0 + e
    r = 1.0 / ope
    lg = jnp.log2(ope)
    pos = t >= 0.9999
    a = jnp.where(pos, lg, x2 + lg)
    omt = 1.0 - t
    d = jnp.where(pos, e, omt * omt)
    loss = a * (r * r) * (d * d)
    posf = jnp.where(pos, 1.0, 0.0)
    return loss, posf


def _tile_sums(x_ref, t_ref, nslices):
    accs = []
    k = 0
    for s in range(nslices):
        for r0 in range(0, _H, _CH):
            rows = slice(r0, r0 + _CH)
            loss, posf = _chunk_terms(x_ref[s, rows, :], t_ref[s, rows, :])
            if k < _N_ACC:
                accs.append([loss, posf])
            else:
                a = accs[k % _N_ACC]
                a[0] += loss
                a[1] += posf
            k += 1
    while len(accs) > 1:
        nxt = []
        for j in range(0, len(accs) - 1, 2):
            nxt.append([accs[j][0] + accs[j + 1][0],
                        accs[j][1] + accs[j + 1][1]])
        if len(accs) % 2:
            nxt.append(accs[-1])
        accs = nxt
    return accs[0]


def _body(ml_ref, mm_ref, vl_ref, vm_ref, out_ref,
          macc_l, macc_p, vacc_l, vacc_p):
    i = pl.program_id(0)
    m_l, m_p = _tile_sums(ml_ref, mm_ref, _MAIN_BLK)
    v_l, v_p = _tile_sums(vl_ref, vm_ref, _VERT_BLK)

    @pl.when(i == 0)
    def _init():
        macc_l[...] = m_l
        macc_p[...] = m_p
        vacc_l[...] = v_l
        vacc_p[...] = v_p

    @pl.when(i > 0)
    def _accum():
        macc_l[...] += m_l
        macc_p[...] += m_p
        vacc_l[...] += v_l
        vacc_p[...] += v_p

    @pl.when(i == _GRID - 1)
    def _finalize():
        ms = jnp.sum(macc_l[...]) * _LN2
        mp = jnp.sum(macc_p[...])
        vs = jnp.sum(vacc_l[...]) * _LN2
        vp = jnp.sum(vacc_p[...])
        main_loss = ms / jnp.maximum(mp, 1.0)
        vert_loss = vs / jnp.maximum(vp, 1.0)
        out_ref[0, 0] = main_loss + vert_loss


def kernel(main_kf_logits, main_kf_mask, vertex_kf_logits, vertex_kf_mask):
    ml = main_kf_logits.reshape(_MAIN_SLICES, _H, _W)
    mm = main_kf_mask.reshape(_MAIN_SLICES, _H, _W)
    vl = vertex_kf_logits.reshape(_VERT_SLICES, _H, _W)
    vm = vertex_kf_mask.reshape(_VERT_SLICES, _H, _W)

    main_spec = pl.BlockSpec((_MAIN_BLK, _H, _W), lambda i: (i, 0, 0))
    vert_spec = pl.BlockSpec((_VERT_BLK, _H, _W), lambda i: (i, 0, 0))

    out = pl.pallas_call(
        _body,
        grid=(_GRID,),
        in_specs=[main_spec, main_spec, vert_spec, vert_spec],
        out_specs=pl.BlockSpec(memory_space=pltpu.SMEM),
        out_shape=jax.ShapeDtypeStruct((1, 1), jnp.float32),
        scratch_shapes=[
            pltpu.VMEM((_CH, _W), jnp.float32),
            pltpu.VMEM((_CH, _W), jnp.float32),
            pltpu.VMEM((_CH, _W), jnp.float32),
            pltpu.VMEM((_CH, _W), jnp.float32),
        ],
        compiler_params=pltpu.CompilerParams(
            dimension_semantics=("arbitrary",),
        ),
    )(ml, mm, vl, vm)
    return out[0, 0]


# speedup vs baseline: 4.0925x; 1.0433x over previous
import jax
import jax.numpy as jnp
from jax.experimental import pallas as pl
from jax.experimental.pallas import tpu as pltpu

_GRID = 8
_H, _W = 96, 320
_CH = 8
_N_ACC = 4
_MAIN_SLICES = 16 * 3
_VERT_SLICES = 16 * 9
_MAIN_BLK = _MAIN_SLICES // _GRID
_VERT_BLK = _VERT_SLICES // _GRID

_LOG2E = 1.4426950408889634
_LN2 = 0.6931471805599453
_CLIP2 = 9.210240366975849 * _LOG2E


def _chunk_terms(x, t):
    x2 = jnp.clip(x * _LOG2E, -_CLIP2, _CLIP2)
    e = jnp.exp2(-x2)
    ope = 1.0 + e
    r = 1.0 / ope
    lg = jnp.log2(ope)
    pos = t >= 0.9999
    a = jnp.where(pos, lg, x2 + lg)
    omt = 1.0 - t
    d = jnp.where(pos, e, omt * omt)
    loss = a * (r * r) * (d * d)
    posf = jnp.where(pos, 1.0, 0.0)
    return loss, posf


def _tile_sums(x_ref, t_ref, nslices):
    accs = []
    k = 0
    for s in range(nslices):
        for r0 in range(0, _H, _CH):
            rows = slice(r0, r0 + _CH)
            loss, posf = _chunk_terms(x_ref[s, rows, :], t_ref[s, rows, :])
            if k < _N_ACC:
                accs.append([loss, posf])
            else:
                a = accs[k % _N_ACC]
                a[0] += loss
                a[1] += posf
            k += 1
    while len(accs) > 1:
        nxt = []
        for j in range(0, len(accs) - 1, 2):
            nxt.append([accs[j][0] + accs[j + 1][0],
                        accs[j][1] + accs[j + 1][1]])
        if len(accs) % 2:
            nxt.append(accs[-1])
        accs = nxt
    return accs[0]


def _body(ml_ref, mm_ref, vl_ref, vm_ref, out_ref,
          macc_l, macc_p, vacc_l, vacc_p):
    i = pl.program_id(0)
    m_l, m_p = _tile_sums(ml_ref, mm_ref, _MAIN_BLK)
    v_l, v_p = _tile_sums(vl_ref, vm_ref, _VERT_BLK)

    @pl.when(i == 0)
    def _init():
        macc_l[...] = m_l
        macc_p[...] = m_p
        vacc_l[...] = v_l
        vacc_p[...] = v_p

    @pl.when(i > 0)
    def _accum():
        macc_l[...] += m_l
        macc_p[...] += m_p
        vacc_l[...] += v_l
        vacc_p[...] += v_p

    @pl.when(i == _GRID - 1)
    def _finalize():
        ms = jnp.sum(macc_l[...]) * _LN2
        mp = jnp.sum(macc_p[...])
        vs = jnp.sum(vacc_l[...]) * _LN2
        vp = jnp.sum(vacc_p[...])
        main_loss = ms / jnp.maximum(mp, 1.0)
        vert_loss = vs / jnp.maximum(vp, 1.0)
        out_ref[0, 0] = main_loss + vert_loss


def kernel(main_kf_logits, main_kf_mask, vertex_kf_logits, vertex_kf_mask):
    ml = main_kf_logits.reshape(_MAIN_SLICES, _H, _W)
    mm = main_kf_mask.reshape(_MAIN_SLICES, _H, _W)
    vl = vertex_kf_logits.reshape(_VERT_SLICES, _H, _W)
    vm = vertex_kf_mask.reshape(_VERT_SLICES, _H, _W)

    main_spec = pl.BlockSpec((_MAIN_BLK, _H, _W), lambda i: (i, 0, 0))
    vert_spec = pl.BlockSpec((_VERT_BLK, _H, _W), lambda i: (i, 0, 0))

    out = pl.pallas_call(
        _body,
        grid=(_GRID,),
        in_specs=[main_spec, main_spec, vert_spec, vert_spec],
        out_specs=pl.BlockSpec(memory_space=pltpu.SMEM),
        out_shape=jax.ShapeDtypeStruct((1, 1), jnp.float32),
        scratch_shapes=[
            pltpu.VMEM((_CH, _W), jnp.float32),
            pltpu.VMEM((_CH, _W), jnp.float32),
            pltpu.VMEM((_CH, _W), jnp.float32),
            pltpu.VMEM((_CH, _W), jnp.float32),
        ],
        compiler_params=pltpu.CompilerParams(
            dimension_semantics=("arbitrary",),
        ),
    )(ml, mm, vl, vm)
    return out[0, 0]
